# baseline (speedup 1.0000x reference)
import jax
import jax.numpy as jnp
from jax.experimental import pallas as pl
from jax.experimental.pallas import tpu as pltpu

_NUM_HEADS = 8


def _pad16(n):
    return -(-n // 16) * 16


def _layernorm(x, g, b, eps=1e-5):
    mu = jnp.mean(x, axis=-1, keepdims=True)
    xc = x - mu
    var = jnp.mean(xc * xc, axis=-1, keepdims=True)
    return xc * jax.lax.rsqrt(var + eps) * g + b


def _proj(x_bf, w, b):
    return jnp.dot(x_bf, w, preferred_element_type=jnp.float32) + b


def _attend_rows(q_tile, k, v, bias, ctx_ref, r0):
    TR, C = q_tile.shape
    hd = C // _NUM_HEADS
    for h in range(_NUM_HEADS):
        sl = slice(h * hd, (h + 1) * hd)
        logits = jax.lax.dot_general(
            q_tile[:, sl], k[:, sl], (((1,), (1,)), ((), ())),
            preferred_element_type=jnp.float32)
        if bias is not None:
            logits = logits + bias
        m = jnp.max(logits, axis=-1, keepdims=True)
        p = jnp.exp(logits - m)
        d = jnp.sum(p, axis=-1, keepdims=True)
        p = p * pl.reciprocal(d, approx=True)
        ctx_ref[r0:r0 + TR, sl] = jnp.dot(p.astype(jnp.bfloat16), v[:, sl],
                                          preferred_element_type=jnp.float32)


def _ffn_tile(x, w1, b1, w2, b2):
    hmid = jnp.maximum(_proj(x.astype(jnp.bfloat16), w1, b1), 0.0)
    return _proj(hmid.astype(jnp.bfloat16), w2, b2)


def _enc_body(x_ref, pos_ref, bias_ref, wqkv_ref, bqkv_ref, wo_ref, bo_ref,
              w1_ref, b1_ref, w2_ref, b2_ref, ln_ref, mem_ref, ctx_ref):
    _, S, C = mem_ref.shape
    TR = 256 if S % 256 == 0 else S
    NT = S // TR

    @pl.when(pl.program_id(1) == 0)
    def _():
        mem_ref[...] = x_ref[...]

    x = mem_ref[0]
    wqkv = wqkv_ref[0]
    bqkv = bqkv_ref[0]
    ln = ln_ref[0]
    bias = bias_ref[0]

    qk_in = (x + pos_ref[0]).astype(jnp.bfloat16)
    qk = _proj(qk_in, wqkv[:, :2 * C], bqkv[:, :2 * C])
    q = qk[:, :C].astype(jnp.bfloat16)
    k = qk[:, C:].astype(jnp.bfloat16)
    v = _proj(x.astype(jnp.bfloat16), wqkv[:, 2 * C:],
              bqkv[:, 2 * C:]).astype(jnp.bfloat16)

    for t in range(NT):
        r0 = t * TR
        _attend_rows(q[r0:r0 + TR], k, v, bias, ctx_ref, r0)
        sa = _proj(ctx_ref[r0:r0 + TR].astype(jnp.bfloat16),
                   wo_ref[0], bo_ref[0])
        x1 = _layernorm(mem_ref[0, r0:r0 + TR, :] + sa, ln[0:1], ln[1:2])
        ffn = _ffn_tile(x1, w1_ref[0], b1_ref[0], w2_ref[0], b2_ref[0])
        mem_ref[0, r0:r0 + TR, :] = _layernorm(x1 + ffn, ln[2:3], ln[3:4])


def _dec_body(mem_ref, qpos_ref, pos_ref, bias_ref,
              sa_wqkv_ref, sa_bqkv_ref, sa_wo_ref, sa_bo_ref,
              ca_wqkv_ref, ca_bqkv_ref, ca_wo_ref, ca_bo_ref,
              w1_ref, b1_ref, w2_ref, b2_ref, ln_ref, dn_ref,
              hid_ref, tgt_ref, ctx_ref, memk_ref, memv_ref):
    Np, C = tgt_ref.shape
    Nq = hid_ref.shape[2]
    S = mem_ref.shape[1]
    TR = Np // 2

    @pl.when(pl.program_id(1) == 0)
    def _():
        tgt_ref[...] = jnp.zeros_like(tgt_ref)
        m = mem_ref[0]
        memk_ref[...] = (m + pos_ref[0]).astype(jnp.bfloat16)
        memv_ref[...] = m.astype(jnp.bfloat16)

    tgt = tgt_ref[...]
    qpos = qpos_ref[...]
    ln = ln_ref[0]
    bias = bias_ref[0]

    pad_bias = jnp.where(
        jax.lax.broadcasted_iota(jnp.int32, (1, Np), 1) >= Nq, -1e9, 0.0)
    wq, bq = sa_wqkv_ref[0], sa_bqkv_ref[0]
    qk = _proj((tgt + qpos).astype(jnp.bfloat16), wq[:, :2 * C], bq[:, :2 * C])
    q = qk[:, :C].astype(jnp.bfloat16)
    k = qk[:, C:].astype(jnp.bfloat16)
    v = _proj(tgt.astype(jnp.bfloat16), wq[:, 2 * C:],
              bq[:, 2 * C:]).astype(jnp.bfloat16)
    _attend_rows(q, k, v, pad_bias, ctx_ref, 0)
    sa = _proj(ctx_ref[...].astype(jnp.bfloat16), sa_wo_ref[0], sa_bo_ref[0])
    tgt = _layernorm(tgt + sa, ln[0:1], ln[1:2])

    wc, bc = ca_wqkv_ref[0], ca_bqkv_ref[0]
    q = _proj((tgt + qpos).astype(jnp.bfloat16), wc[:, :C],
              bc[:, :C]).astype(jnp.bfloat16)
    k = _proj(memk_ref[...], wc[:, C:2 * C],
              bc[:, C:2 * C]).astype(jnp.bfloat16)
    v = _proj(memv_ref[...], wc[:, 2 * C:],
              bc[:, 2 * C:]).astype(jnp.bfloat16)

    for t in range(2):
        r0 = t * TR
        _attend_rows(q[r0:r0 + TR], k, v, bias, ctx_ref, r0)
        ca = _proj(ctx_ref[r0:r0 + TR].astype(jnp.bfloat16),
                   ca_wo_ref[0], ca_bo_ref[0])
        t1 = _layernorm(tgt[r0:r0 + TR] + ca, ln[2:3], ln[3:4])
        ffn = _ffn_tile(t1, w1_ref[0], b1_ref[0], w2_ref[0], b2_ref[0])
        tgt_ref[r0:r0 + TR, :] = _layernorm(t1 + ffn, ln[4:5], ln[5:6])

    dn = dn_ref[...]
    hid_ref[0, 0] = _layernorm(tgt_ref[...], dn[0:1], dn[1:2])[:Nq]


def kernel(x, mask, query_embed, pos_embed,
           e_wqkv, e_bqkv, e_wo, e_bo, e_ffn_w1, e_ffn_b1, e_ffn_w2,
           e_ffn_b2, e_ln,
           d_sa_wqkv, d_sa_bqkv, d_sa_wo, d_sa_bo,
           d_ca_wqkv, d_ca_bqkv, d_ca_wo, d_ca_bo,
           d_ffn_w1, d_ffn_b1, d_ffn_w2, d_ffn_b2, d_ln,
           dec_norm, ref_w1, ref_b1, ref_w2, ref_b2):
    B, C, hh, ww = x.shape
    S = hh * ww
    Nq = query_embed.shape[0]
    Np = _pad16(Nq)
    Le, F = e_ffn_w1.shape[0], e_ffn_w1.shape[-1]
    Ld = d_ffn_w1.shape[0]
    NH = _NUM_HEADS

    xt = x.reshape(B, C, S).transpose(0, 2, 1)
    post = pos_embed.reshape(B, C, S).transpose(0, 2, 1)
    key_bias = jnp.where(mask.reshape(B, S), -1e9, 0.0).astype(jnp.float32)
    key_bias = key_bias.reshape(B, 1, S)
    qe_pad = jnp.pad(query_embed, ((0, Np - Nq), (0, 0)))

    bspec = lambda shape: pl.BlockSpec((1,) + shape, lambda b, l: (b, 0, 0))
    wspec = lambda shape: pl.BlockSpec((1,) + shape, lambda b, l: (l, 0, 0))

    e_flops = Le * B * (8 * S * C * C + 4 * S * S * C + 4 * S * C * F)
    e_trans = Le * B * NH * (S * S + S)
    e_wbytes = 2 * (4 * C * C + 2 * C * F) + 4 * (5 * C + F + 4 * C)
    e_bytes = 4 * B * (3 * S * C + S) + B * Le * e_wbytes

    memory = pl.pallas_call(
        _enc_body,
        out_shape=jax.ShapeDtypeStruct((B, S, C), jnp.float32),
        grid=(B, Le),
        in_specs=[
            bspec((S, C)),
            bspec((S, C)),
            bspec((1, S)),
            wspec((C, 3 * C)), wspec((1, 3 * C)), wspec((C, C)), wspec((1, C)),
            wspec((C, F)), wspec((1, F)), wspec((F, C)), wspec((1, C)),
            wspec((4, C)),
        ],
        out_specs=bspec((S, C)),
        scratch_shapes=[pltpu.VMEM((S, C), jnp.float32)],
        compiler_params=pltpu.CompilerParams(
            dimension_semantics=("arbitrary", "arbitrary")),
        cost_estimate=pl.CostEstimate(flops=e_flops, transcendentals=e_trans,
                                      bytes_accessed=e_bytes),
    )(xt, post, key_bias,
      e_wqkv, e_bqkv, e_wo, e_bo,
      e_ffn_w1, e_ffn_b1, e_ffn_w2, e_ffn_b2, e_ln)

    d_flops = Ld * B * (12 * Nq * C * C + 4 * S * C * C + 4 * Nq * Nq * C
                        + 4 * Nq * S * C + 4 * Nq * C * F)
    d_trans = Ld * B * NH * (Nq * Nq + Nq * S + 2 * Nq)
    d_wbytes = 2 * (8 * C * C + 2 * C * F) + 4 * (10 * C + F + 8 * C)
    d_bytes = (4 * B * (2 * S * C + Nq * C + S) + 4 * Ld * B * Nq * C
               + B * Ld * d_wbytes)

    hid = pl.pallas_call(
        _dec_body,
        out_shape=jax.ShapeDtypeStruct((Ld, B, Nq, C), jnp.float32),
        grid=(B, Ld),
        in_specs=[
            bspec((S, C)),
            pl.BlockSpec((Np, C), lambda b, l: (0, 0)),
            bspec((S, C)),
            bspec((1, S)),
            wspec((C, 3 * C)), wspec((1, 3 * C)), wspec((C, C)), wspec((1, C)),
            wspec((C, 3 * C)), wspec((1, 3 * C)), wspec((C, C)), wspec((1, C)),
            wspec((C, F)), wspec((1, F)), wspec((F, C)), wspec((1, C)),
            wspec((6, C)),
            pl.BlockSpec((2, C), lambda b, l: (0, 0)),
        ],
        out_specs=pl.BlockSpec((1, 1, Nq, C), lambda b, l: (l, b, 0, 0)),
        scratch_shapes=[
            pltpu.VMEM((Np, C), jnp.float32),
            pltpu.VMEM((Np, C), jnp.float32),
            pltpu.VMEM((S, C), jnp.bfloat16),
            pltpu.VMEM((S, C), jnp.bfloat16),
        ],
        compiler_params=pltpu.CompilerParams(
            dimension_semantics=("arbitrary", "arbitrary")),
        cost_estimate=pl.CostEstimate(flops=d_flops, transcendentals=d_trans,
                                      bytes_accessed=d_bytes),
    )(memory, qe_pad, post, key_bias,
      d_sa_wqkv, d_sa_bqkv, d_sa_wo, d_sa_bo,
      d_ca_wqkv, d_ca_bqkv, d_ca_wo, d_ca_bo,
      d_ffn_w1, d_ffn_b1, d_ffn_w2, d_ffn_b2, d_ln, dec_norm)

    hidden_state = jnp.transpose(hid, (0, 2, 1, 3))

    query_pos = jnp.broadcast_to(query_embed[None], (B, Nq, C))
    r = jnp.maximum(query_pos.reshape(B * Nq, C) @ ref_w1 + ref_b1, 0.0)
    r = r @ ref_w2 + ref_b2
    references = jax.nn.sigmoid(r).reshape(B, Nq, 2)

    return hidden_state, references


# speedup vs baseline: 1.8181x; 1.8181x over previous
import jax
import jax.numpy as jnp
from jax.experimental import pallas as pl
from jax.experimental.pallas import tpu as pltpu

_NUM_HEADS = 8


def _layernorm(x, g, b, eps=1e-5):
    mu = jnp.mean(x, axis=-1, keepdims=True)
    xc = x - mu
    var = jnp.mean(xc * xc, axis=-1, keepdims=True)
    return xc * jax.lax.rsqrt(var + eps) * g + b


def _proj(x_bf, w, b):
    return jnp.dot(x_bf, w, preferred_element_type=jnp.float32) + b


def _heads_attend(q, k, v, bias, ctx_ref):
    C = q.shape[-1]
    hd = C // _NUM_HEADS
    for h in range(_NUM_HEADS):
        sl = slice(h * hd, (h + 1) * hd)
        logits = jax.lax.dot_general(
            q[:, sl], k[:, sl], (((1,), (1,)), ((), ())),
            preferred_element_type=jnp.float32)
        if bias is not None:
            logits = logits + bias
        p = jnp.exp(logits)
        d = jnp.sum(p, axis=-1, keepdims=True)
        ctx = jnp.dot(p.astype(jnp.bfloat16), v[:, sl],
                      preferred_element_type=jnp.float32)
        ctx_ref[:, sl] = ctx * pl.reciprocal(d, approx=True)


def _enc_body(x_ref, pos_ref, bias_ref, wqkv_ref, bqkv_ref, wo_ref, bo_ref,
              w1_ref, b1_ref, w2_ref, b2_ref, ln_ref, mem_ref, ctx_ref):
    C = mem_ref.shape[-1]

    @pl.when(pl.program_id(1) == 0)
    def _():
        mem_ref[...] = x_ref[...]

    x = mem_ref[0]
    wqkv = wqkv_ref[0]
    bqkv = bqkv_ref[0]
    ln = ln_ref[0]
    bias = bias_ref[0]

    qk_in = (x + pos_ref[0]).astype(jnp.bfloat16)
    qk = _proj(qk_in, wqkv[:, :2 * C], bqkv[:, :2 * C])
    q = qk[:, :C].astype(jnp.bfloat16)
    k = qk[:, C:].astype(jnp.bfloat16)
    v = _proj(x.astype(jnp.bfloat16), wqkv[:, 2 * C:],
              bqkv[:, 2 * C:]).astype(jnp.bfloat16)

    _heads_attend(q, k, v, bias, ctx_ref)
    sa = _proj(ctx_ref[...].astype(jnp.bfloat16), wo_ref[0], bo_ref[0])
    x = _layernorm(x + sa, ln[0:1], ln[1:2])

    hmid = jnp.maximum(_proj(x.astype(jnp.bfloat16), w1_ref[0], b1_ref[0]),
                       0.0)
    ffn = _proj(hmid.astype(jnp.bfloat16), w2_ref[0], b2_ref[0])
    x = _layernorm(x + ffn, ln[2:3], ln[3:4])

    mem_ref[...] = x[None]


def _dec_body(mem_ref, qpos_ref, pos_ref, bias_ref,
              sa_wqkv_ref, sa_bqkv_ref, sa_wo_ref, sa_bo_ref,
              ca_wqkv_ref, ca_bqkv_ref, ca_wo_ref, ca_bo_ref,
              w1_ref, b1_ref, w2_ref, b2_ref, ln_ref, dn_ref,
              hid_ref, tgt_ref, ctx_ref, memk_ref, memv_ref):
    C = tgt_ref.shape[-1]

    @pl.when(pl.program_id(1) == 0)
    def _():
        tgt_ref[...] = jnp.zeros_like(tgt_ref)
        m = mem_ref[0]
        memk_ref[...] = (m + pos_ref[0]).astype(jnp.bfloat16)
        memv_ref[...] = m.astype(jnp.bfloat16)

    tgt = tgt_ref[...]
    qpos = qpos_ref[...]
    ln = ln_ref[0]
    bias = bias_ref[0]

    wq, bq = sa_wqkv_ref[0], sa_bqkv_ref[0]
    qk = _proj((tgt + qpos).astype(jnp.bfloat16), wq[:, :2 * C], bq[:, :2 * C])
    q = qk[:, :C].astype(jnp.bfloat16)
    k = qk[:, C:].astype(jnp.bfloat16)
    v = _proj(tgt.astype(jnp.bfloat16), wq[:, 2 * C:],
              bq[:, 2 * C:]).astype(jnp.bfloat16)
    _heads_attend(q, k, v, None, ctx_ref)
    sa = _proj(ctx_ref[...].astype(jnp.bfloat16), sa_wo_ref[0], sa_bo_ref[0])
    tgt = _layernorm(tgt + sa, ln[0:1], ln[1:2])

    wc, bc = ca_wqkv_ref[0], ca_bqkv_ref[0]
    q = _proj((tgt + qpos).astype(jnp.bfloat16), wc[:, :C],
              bc[:, :C]).astype(jnp.bfloat16)
    k = _proj(memk_ref[...], wc[:, C:2 * C],
              bc[:, C:2 * C]).astype(jnp.bfloat16)
    v = _proj(memv_ref[...], wc[:, 2 * C:],
              bc[:, 2 * C:]).astype(jnp.bfloat16)
    _heads_attend(q, k, v, bias, ctx_ref)
    ca = _proj(ctx_ref[...].astype(jnp.bfloat16), ca_wo_ref[0], ca_bo_ref[0])
    tgt = _layernorm(tgt + ca, ln[2:3], ln[3:4])

    hmid = jnp.maximum(_proj(tgt.astype(jnp.bfloat16), w1_ref[0], b1_ref[0]),
                       0.0)
    ffn = _proj(hmid.astype(jnp.bfloat16), w2_ref[0], b2_ref[0])
    tgt = _layernorm(tgt + ffn, ln[4:5], ln[5:6])

    tgt_ref[...] = tgt
    dn = dn_ref[...]
    hid_ref[0, 0] = _layernorm(tgt, dn[0:1], dn[1:2])


def kernel(x, mask, query_embed, pos_embed,
           e_wqkv, e_bqkv, e_wo, e_bo, e_ffn_w1, e_ffn_b1, e_ffn_w2,
           e_ffn_b2, e_ln,
           d_sa_wqkv, d_sa_bqkv, d_sa_wo, d_sa_bo,
           d_ca_wqkv, d_ca_bqkv, d_ca_wo, d_ca_bo,
           d_ffn_w1, d_ffn_b1, d_ffn_w2, d_ffn_b2, d_ln,
           dec_norm, ref_w1, ref_b1, ref_w2, ref_b2):
    B, C, hh, ww = x.shape
    S = hh * ww
    Nq = query_embed.shape[0]
    Le, F = e_ffn_w1.shape[0], e_ffn_w1.shape[-1]
    Ld = d_ffn_w1.shape[0]
    NH = _NUM_HEADS

    xt = x.reshape(B, C, S).transpose(0, 2, 1)
    post = pos_embed.reshape(B, C, S).transpose(0, 2, 1)
    key_bias = jnp.where(mask.reshape(B, S), -1e9, 0.0).astype(jnp.float32)
    key_bias = key_bias.reshape(B, 1, S)

    bspec = lambda shape: pl.BlockSpec((1,) + shape, lambda b, l: (b, 0, 0))
    wspec = lambda shape: pl.BlockSpec((1,) + shape, lambda b, l: (l, 0, 0))

    e_flops = Le * B * (8 * S * C * C + 4 * S * S * C + 4 * S * C * F)
    e_trans = Le * B * NH * (S * S + S)
    e_wbytes = 2 * (4 * C * C + 2 * C * F) + 4 * (5 * C + F + 4 * C)
    e_bytes = 4 * B * (3 * S * C + S) + B * Le * e_wbytes

    memory = pl.pallas_call(
        _enc_body,
        out_shape=jax.ShapeDtypeStruct((B, S, C), jnp.float32),
        grid=(B, Le),
        in_specs=[
            bspec((S, C)),
            bspec((S, C)),
            bspec((1, S)),
            wspec((C, 3 * C)), wspec((1, 3 * C)), wspec((C, C)), wspec((1, C)),
            wspec((C, F)), wspec((1, F)), wspec((F, C)), wspec((1, C)),
            wspec((4, C)),
        ],
        out_specs=bspec((S, C)),
        scratch_shapes=[pltpu.VMEM((S, C), jnp.float32)],
        compiler_params=pltpu.CompilerParams(
            dimension_semantics=("arbitrary", "arbitrary")),
        cost_estimate=pl.CostEstimate(flops=e_flops, transcendentals=e_trans,
                                      bytes_accessed=e_bytes),
    )(xt, post, key_bias,
      e_wqkv, e_bqkv, e_wo, e_bo,
      e_ffn_w1, e_ffn_b1, e_ffn_w2, e_ffn_b2, e_ln)

    d_flops = Ld * B * (12 * Nq * C * C + 4 * S * C * C + 4 * Nq * Nq * C
                        + 4 * Nq * S * C + 4 * Nq * C * F)
    d_trans = Ld * B * NH * (Nq * Nq + Nq * S + 2 * Nq)
    d_wbytes = 2 * (8 * C * C + 2 * C * F) + 4 * (10 * C + F + 8 * C)
    d_bytes = (4 * B * (2 * S * C + Nq * C + S) + 4 * Ld * B * Nq * C
               + B * Ld * d_wbytes)

    hid = pl.pallas_call(
        _dec_body,
        out_shape=jax.ShapeDtypeStruct((Ld, B, Nq, C), jnp.float32),
        grid=(B, Ld),
        in_specs=[
            bspec((S, C)),
            pl.BlockSpec((Nq, C), lambda b, l: (0, 0)),
            bspec((S, C)),
            bspec((1, S)),
            wspec((C, 3 * C)), wspec((1, 3 * C)), wspec((C, C)), wspec((1, C)),
            wspec((C, 3 * C)), wspec((1, 3 * C)), wspec((C, C)), wspec((1, C)),
            wspec((C, F)), wspec((1, F)), wspec((F, C)), wspec((1, C)),
            wspec((6, C)),
            pl.BlockSpec((2, C), lambda b, l: (0, 0)),
        ],
        out_specs=pl.BlockSpec((1, 1, Nq, C), lambda b, l: (l, b, 0, 0)),
        scratch_shapes=[
            pltpu.VMEM((Nq, C), jnp.float32),
            pltpu.VMEM((Nq, C), jnp.float32),
            pltpu.VMEM((S, C), jnp.bfloat16),
            pltpu.VMEM((S, C), jnp.bfloat16),
        ],
        compiler_params=pltpu.CompilerParams(
            dimension_semantics=("arbitrary", "arbitrary")),
        cost_estimate=pl.CostEstimate(flops=d_flops, transcendentals=d_trans,
                                      bytes_accessed=d_bytes),
    )(memory, query_embed, post, key_bias,
      d_sa_wqkv, d_sa_bqkv, d_sa_wo, d_sa_bo,
      d_ca_wqkv, d_ca_bqkv, d_ca_wo, d_ca_bo,
      d_ffn_w1, d_ffn_b1, d_ffn_w2, d_ffn_b2, d_ln, dec_norm)

    hidden_state = jnp.transpose(hid, (0, 2, 1, 3))

    query_pos = jnp.broadcast_to(query_embed[None], (B, Nq, C))
    r = jnp.maximum(query_pos.reshape(B * Nq, C) @ ref_w1 + ref_b1, 0.0)
    r = r @ ref_w2 + ref_b2
    references = jax.nn.sigmoid(r).reshape(B, Nq, 2)

    return hidden_state, references


# speedup vs baseline: 2.0023x; 1.1013x over previous
import jax
import jax.numpy as jnp
from jax.experimental import pallas as pl
from jax.experimental.pallas import tpu as pltpu

_NUM_HEADS = 8


def _layernorm(x, g, b, eps=1e-5):
    mu = jnp.mean(x, axis=-1, keepdims=True)
    xc = x - mu
    var = jnp.mean(xc * xc, axis=-1, keepdims=True)
    return xc * jax.lax.rsqrt(var + eps) * g + b


def _proj(x_bf, w, b):
    return jnp.dot(x_bf, w, preferred_element_type=jnp.float32) + b


def _heads_attend(q, k, v, m_col, ctx_ref):
    C = q.shape[-1]
    hd = C // _NUM_HEADS
    for h in range(_NUM_HEADS):
        sl = slice(h * hd, (h + 1) * hd)
        logits = jax.lax.dot_general(
            q[:, sl], k[:, sl], (((1,), (1,)), ((), ())),
            preferred_element_type=jnp.float32)
        p = jnp.exp(logits).astype(jnp.bfloat16)
        vaug = jnp.concatenate([v[:, sl], m_col], axis=-1)
        out = jnp.dot(p, vaug, preferred_element_type=jnp.float32)
        ctx_ref[:, sl] = out[:, :hd] * pl.reciprocal(out[:, hd:hd + 1],
                                                     approx=True)


def _enc_body(x_ref, pos_ref, valid_ref, wqkv_ref, bqkv_ref, wo_ref, bo_ref,
              w1_ref, b1_ref, w2_ref, b2_ref, ln_ref, mem_ref, ctx_ref):
    C = mem_ref.shape[-1]

    @pl.when(pl.program_id(1) == 0)
    def _():
        mem_ref[...] = x_ref[...]

    x = mem_ref[0]
    wqkv = wqkv_ref[0]
    bqkv = bqkv_ref[0]
    ln = ln_ref[0]
    m_col = valid_ref[0][:, 0:1]

    qk_in = (x + pos_ref[0]).astype(jnp.bfloat16)
    qk = _proj(qk_in, wqkv[:, :2 * C], bqkv[:, :2 * C])
    q = qk[:, :C].astype(jnp.bfloat16)
    k = qk[:, C:].astype(jnp.bfloat16)
    v = _proj(x.astype(jnp.bfloat16), wqkv[:, 2 * C:],
              bqkv[:, 2 * C:]).astype(jnp.bfloat16) * m_col

    _heads_attend(q, k, v, m_col, ctx_ref)
    sa = _proj(ctx_ref[...].astype(jnp.bfloat16), wo_ref[0], bo_ref[0])
    x = _layernorm(x + sa, ln[0:1], ln[1:2])

    hmid = jnp.maximum(_proj(x.astype(jnp.bfloat16), w1_ref[0], b1_ref[0]),
                       0.0)
    ffn = _proj(hmid.astype(jnp.bfloat16), w2_ref[0], b2_ref[0])
    x = _layernorm(x + ffn, ln[2:3], ln[3:4])

    mem_ref[...] = x[None]


def _dec_body(mem_ref, qpos_ref, pos_ref, valid_ref,
              sa_wqkv_ref, sa_bqkv_ref, sa_wo_ref, sa_bo_ref,
              ca_wqkv_ref, ca_bqkv_ref, ca_wo_ref, ca_bo_ref,
              w1_ref, b1_ref, w2_ref, b2_ref, ln_ref, dn_ref,
              hid_ref, tgt_ref, ctx_ref, memk_ref, memv_ref):
    Nq, C = tgt_ref.shape

    @pl.when(pl.program_id(1) == 0)
    def _():
        tgt_ref[...] = jnp.zeros_like(tgt_ref)
        m = mem_ref[0]
        memk_ref[...] = (m + pos_ref[0]).astype(jnp.bfloat16)
        memv_ref[...] = m.astype(jnp.bfloat16)

    tgt = tgt_ref[...]
    qpos = qpos_ref[...]
    ln = ln_ref[0]
    m_col = valid_ref[0][:, 0:1]
    ones_col = jnp.ones((Nq, 1), jnp.bfloat16)

    wq, bq = sa_wqkv_ref[0], sa_bqkv_ref[0]
    qk = _proj((tgt + qpos).astype(jnp.bfloat16), wq[:, :2 * C], bq[:, :2 * C])
    q = qk[:, :C].astype(jnp.bfloat16)
    k = qk[:, C:].astype(jnp.bfloat16)
    v = _proj(tgt.astype(jnp.bfloat16), wq[:, 2 * C:],
              bq[:, 2 * C:]).astype(jnp.bfloat16)
    _heads_attend(q, k, v, ones_col, ctx_ref)
    sa = _proj(ctx_ref[...].astype(jnp.bfloat16), sa_wo_ref[0], sa_bo_ref[0])
    tgt = _layernorm(tgt + sa, ln[0:1], ln[1:2])

    wc, bc = ca_wqkv_ref[0], ca_bqkv_ref[0]
    q = _proj((tgt + qpos).astype(jnp.bfloat16), wc[:, :C],
              bc[:, :C]).astype(jnp.bfloat16)
    k = _proj(memk_ref[...], wc[:, C:2 * C],
              bc[:, C:2 * C]).astype(jnp.bfloat16)
    v = _proj(memv_ref[...], wc[:, 2 * C:],
              bc[:, 2 * C:]).astype(jnp.bfloat16) * m_col
    _heads_attend(q, k, v, m_col, ctx_ref)
    ca = _proj(ctx_ref[...].astype(jnp.bfloat16), ca_wo_ref[0], ca_bo_ref[0])
    tgt = _layernorm(tgt + ca, ln[2:3], ln[3:4])

    hmid = jnp.maximum(_proj(tgt.astype(jnp.bfloat16), w1_ref[0], b1_ref[0]),
                       0.0)
    ffn = _proj(hmid.astype(jnp.bfloat16), w2_ref[0], b2_ref[0])
    tgt = _layernorm(tgt + ffn, ln[4:5], ln[5:6])

    tgt_ref[...] = tgt
    dn = dn_ref[...]
    hid_ref[0, 0] = _layernorm(tgt, dn[0:1], dn[1:2])


def kernel(x, mask, query_embed, pos_embed,
           e_wqkv, e_bqkv, e_wo, e_bo, e_ffn_w1, e_ffn_b1, e_ffn_w2,
           e_ffn_b2, e_ln,
           d_sa_wqkv, d_sa_bqkv, d_sa_wo, d_sa_bo,
           d_ca_wqkv, d_ca_bqkv, d_ca_wo, d_ca_bo,
           d_ffn_w1, d_ffn_b1, d_ffn_w2, d_ffn_b2, d_ln,
           dec_norm, ref_w1, ref_b1, ref_w2, ref_b2):
    B, C, hh, ww = x.shape
    S = hh * ww
    Nq = query_embed.shape[0]
    Le, F = e_ffn_w1.shape[0], e_ffn_w1.shape[-1]
    Ld = d_ffn_w1.shape[0]
    NH = _NUM_HEADS

    xt = x.reshape(B, C, S).transpose(0, 2, 1)
    post = pos_embed.reshape(B, C, S).transpose(0, 2, 1)
    valid = jnp.broadcast_to(
        jnp.where(mask.reshape(B, S), 0.0, 1.0).astype(jnp.bfloat16)[..., None],
        (B, S, 128))

    bspec = lambda shape: pl.BlockSpec((1,) + shape, lambda b, l: (b, 0, 0))
    wspec = lambda shape: pl.BlockSpec((1,) + shape, lambda b, l: (l, 0, 0))

    e_flops = Le * B * (8 * S * C * C + 4 * S * S * C + 4 * S * C * F)
    e_trans = Le * B * NH * (S * S + S)
    e_wbytes = 2 * (4 * C * C + 2 * C * F) + 4 * (5 * C + F + 4 * C)
    e_bytes = 4 * B * (3 * S * C + S) + B * Le * e_wbytes

    memory = pl.pallas_call(
        _enc_body,
        out_shape=jax.ShapeDtypeStruct((B, S, C), jnp.float32),
        grid=(B, Le),
        in_specs=[
            bspec((S, C)),
            bspec((S, C)),
            bspec((S, 128)),
            wspec((C, 3 * C)), wspec((1, 3 * C)), wspec((C, C)), wspec((1, C)),
            wspec((C, F)), wspec((1, F)), wspec((F, C)), wspec((1, C)),
            wspec((4, C)),
        ],
        out_specs=bspec((S, C)),
        scratch_shapes=[pltpu.VMEM((S, C), jnp.float32)],
        compiler_params=pltpu.CompilerParams(
            dimension_semantics=("arbitrary", "arbitrary")),
        cost_estimate=pl.CostEstimate(flops=e_flops, transcendentals=e_trans,
                                      bytes_accessed=e_bytes),
    )(xt, post, valid,
      e_wqkv, e_bqkv, e_wo, e_bo,
      e_ffn_w1, e_ffn_b1, e_ffn_w2, e_ffn_b2, e_ln)

    d_flops = Ld * B * (12 * Nq * C * C + 4 * S * C * C + 4 * Nq * Nq * C
                        + 4 * Nq * S * C + 4 * Nq * C * F)
    d_trans = Ld * B * NH * (Nq * Nq + Nq * S + 2 * Nq)
    d_wbytes = 2 * (8 * C * C + 2 * C * F) + 4 * (10 * C + F + 8 * C)
    d_bytes = (4 * B * (2 * S * C + Nq * C + S) + 4 * Ld * B * Nq * C
               + B * Ld * d_wbytes)

    hid = pl.pallas_call(
        _dec_body,
        out_shape=jax.ShapeDtypeStruct((Ld, B, Nq, C), jnp.float32),
        grid=(B, Ld),
        in_specs=[
            bspec((S, C)),
            pl.BlockSpec((Nq, C), lambda b, l: (0, 0)),
            bspec((S, C)),
            bspec((S, 128)),
            wspec((C, 3 * C)), wspec((1, 3 * C)), wspec((C, C)), wspec((1, C)),
            wspec((C, 3 * C)), wspec((1, 3 * C)), wspec((C, C)), wspec((1, C)),
            wspec((C, F)), wspec((1, F)), wspec((F, C)), wspec((1, C)),
            wspec((6, C)),
            pl.BlockSpec((2, C), lambda b, l: (0, 0)),
        ],
        out_specs=pl.BlockSpec((1, 1, Nq, C), lambda b, l: (l, b, 0, 0)),
        scratch_shapes=[
            pltpu.VMEM((Nq, C), jnp.float32),
            pltpu.VMEM((Nq, C), jnp.float32),
            pltpu.VMEM((S, C), jnp.bfloat16),
            pltpu.VMEM((S, C), jnp.bfloat16),
        ],
        compiler_params=pltpu.CompilerParams(
            dimension_semantics=("arbitrary", "arbitrary")),
        cost_estimate=pl.CostEstimate(flops=d_flops, transcendentals=d_trans,
                                      bytes_accessed=d_bytes),
    )(memory, query_embed, post, valid,
      d_sa_wqkv, d_sa_bqkv, d_sa_wo, d_sa_bo,
      d_ca_wqkv, d_ca_bqkv, d_ca_wo, d_ca_bo,
      d_ffn_w1, d_ffn_b1, d_ffn_w2, d_ffn_b2, d_ln, dec_norm)

    hidden_state = jnp.transpose(hid, (0, 2, 1, 3))

    query_pos = jnp.broadcast_to(query_embed[None], (B, Nq, C))
    r = jnp.maximum(query_pos.reshape(B * Nq, C) @ ref_w1 + ref_b1, 0.0)
    r = r @ ref_w2 + ref_b2
    references = jax.nn.sigmoid(r).reshape(B, Nq, 2)

    return hidden_state, references


# speedup vs baseline: 2.0026x; 1.0001x over previous
import jax
import jax.numpy as jnp
from jax.experimental import pallas as pl
from jax.experimental.pallas import tpu as pltpu

_NUM_HEADS = 8


def _layernorm(x, g, b, eps=1e-5):
    mu = jnp.mean(x, axis=-1, keepdims=True)
    xc = x - mu
    var = jnp.mean(xc * xc, axis=-1, keepdims=True)
    return xc * jax.lax.rsqrt(var + eps) * g + b


def _proj(x_bf, w, b):
    return jnp.dot(x_bf, w, preferred_element_type=jnp.float32) + b


def _heads_attend(q, k, v, m_col, ctx_ref):
    Sq = q.shape[0]
    C = q.shape[-1]
    hd = C // _NUM_HEADS
    for h in range(_NUM_HEADS):
        sl = slice(h * hd, (h + 1) * hd)
        logits = jax.lax.dot_general(
            q[:, sl], k[:, sl], (((1,), (1,)), ((), ())),
            preferred_element_type=jnp.float32)
        p = jnp.exp(logits).astype(jnp.bfloat16)
        vaug = jnp.concatenate([v[:, sl], m_col], axis=-1)
        out = jnp.dot(p, vaug, preferred_element_type=jnp.float32)
        ctx_ref[:Sq, sl] = out[:, :hd] * pl.reciprocal(out[:, hd:hd + 1],
                                                       approx=True)


def _body(x_ref, pos_ref, valid_ref, qe_ref,
          e_wqkv_ref, e_bqkv_ref, e_wo_ref, e_bo_ref,
          e_w1_ref, e_b1_ref, e_w2_ref, e_b2_ref, e_ln_ref,
          sa_wqkv_ref, sa_bqkv_ref, sa_wo_ref, sa_bo_ref,
          ca_wqkv_ref, ca_bqkv_ref, ca_wo_ref, ca_bo_ref,
          d_w1_ref, d_b1_ref, d_w2_ref, d_b2_ref, d_ln_ref, dn_ref,
          hid_ref, mem_ref, tgt_ref, ctx_ref, memk_ref, memv_ref, *, n_enc):
    S, C = mem_ref.shape
    Nq = tgt_ref.shape[0]
    l = pl.program_id(1)

    @pl.when(l == 0)
    def _():
        mem_ref[...] = x_ref[0]

    m_col = valid_ref[0][:, 0:1]

    @pl.when(l < n_enc)
    def _encoder_layer():
        x = mem_ref[...]
        wqkv = e_wqkv_ref[0]
        bqkv = e_bqkv_ref[0]
        ln = e_ln_ref[0]

        qk_in = (x + pos_ref[0]).astype(jnp.bfloat16)
        qk = _proj(qk_in, wqkv[:, :2 * C], bqkv[:, :2 * C])
        q = qk[:, :C].astype(jnp.bfloat16)
        k = qk[:, C:].astype(jnp.bfloat16)
        v = _proj(x.astype(jnp.bfloat16), wqkv[:, 2 * C:],
                  bqkv[:, 2 * C:]).astype(jnp.bfloat16) * m_col

        _heads_attend(q, k, v, m_col, ctx_ref)
        sa = _proj(ctx_ref[...].astype(jnp.bfloat16), e_wo_ref[0], e_bo_ref[0])
        x = _layernorm(x + sa, ln[0:1], ln[1:2])

        hmid = jnp.maximum(
            _proj(x.astype(jnp.bfloat16), e_w1_ref[0], e_b1_ref[0]), 0.0)
        ffn = _proj(hmid.astype(jnp.bfloat16), e_w2_ref[0], e_b2_ref[0])
        mem_ref[...] = _layernorm(x + ffn, ln[2:3], ln[3:4])

    @pl.when(l == n_enc)
    def _decoder_prep():
        tgt_ref[...] = jnp.zeros_like(tgt_ref)
        m = mem_ref[...]
        memk_ref[...] = (m + pos_ref[0]).astype(jnp.bfloat16)
        memv_ref[...] = m.astype(jnp.bfloat16) * m_col

    @pl.when(l >= n_enc)
    def _decoder_layer():
        tgt = tgt_ref[...]
        qpos = qe_ref[...]
        ln = d_ln_ref[0]
        ones_col = jnp.ones((Nq, 1), jnp.bfloat16)

        wq, bq = sa_wqkv_ref[0], sa_bqkv_ref[0]
        qk = _proj((tgt + qpos).astype(jnp.bfloat16),
                   wq[:, :2 * C], bq[:, :2 * C])
        q = qk[:, :C].astype(jnp.bfloat16)
        k = qk[:, C:].astype(jnp.bfloat16)
        v = _proj(tgt.astype(jnp.bfloat16), wq[:, 2 * C:],
                  bq[:, 2 * C:]).astype(jnp.bfloat16)
        _heads_attend(q, k, v, ones_col, ctx_ref)
        sa = _proj(ctx_ref[:Nq].astype(jnp.bfloat16),
                   sa_wo_ref[0], sa_bo_ref[0])
        tgt = _layernorm(tgt + sa, ln[0:1], ln[1:2])

        wc, bc = ca_wqkv_ref[0], ca_bqkv_ref[0]
        q = _proj((tgt + qpos).astype(jnp.bfloat16), wc[:, :C],
                  bc[:, :C]).astype(jnp.bfloat16)
        k = _proj(memk_ref[...], wc[:, C:2 * C],
                  bc[:, C:2 * C]).astype(jnp.bfloat16)
        v = _proj(memv_ref[...], wc[:, 2 * C:],
                  bc[:, 2 * C:]).astype(jnp.bfloat16) * m_col
        _heads_attend(q, k, v, m_col, ctx_ref)
        ca = _proj(ctx_ref[:Nq].astype(jnp.bfloat16),
                   ca_wo_ref[0], ca_bo_ref[0])
        tgt = _layernorm(tgt + ca, ln[2:3], ln[3:4])

        hmid = jnp.maximum(
            _proj(tgt.astype(jnp.bfloat16), d_w1_ref[0], d_b1_ref[0]), 0.0)
        ffn = _proj(hmid.astype(jnp.bfloat16), d_w2_ref[0], d_b2_ref[0])
        tgt = _layernorm(tgt + ffn, ln[4:5], ln[5:6])

        tgt_ref[...] = tgt
        dn = dn_ref[...]
        hid_ref[0, 0] = _layernorm(tgt, dn[0:1], dn[1:2])


def kernel(x, mask, query_embed, pos_embed,
           e_wqkv, e_bqkv, e_wo, e_bo, e_ffn_w1, e_ffn_b1, e_ffn_w2,
           e_ffn_b2, e_ln,
           d_sa_wqkv, d_sa_bqkv, d_sa_wo, d_sa_bo,
           d_ca_wqkv, d_ca_bqkv, d_ca_wo, d_ca_bo,
           d_ffn_w1, d_ffn_b1, d_ffn_w2, d_ffn_b2, d_ln,
           dec_norm, ref_w1, ref_b1, ref_w2, ref_b2):
    from functools import partial

    B, C, hh, ww = x.shape
    S = hh * ww
    Nq = query_embed.shape[0]
    Le, F = e_ffn_w1.shape[0], e_ffn_w1.shape[-1]
    Ld = d_ffn_w1.shape[0]
    NH = _NUM_HEADS
    L = Le + Ld

    xt = x.reshape(B, C, S).transpose(0, 2, 1)
    post = pos_embed.reshape(B, C, S).transpose(0, 2, 1)
    valid = jnp.broadcast_to(
        jnp.where(mask.reshape(B, S), 0.0, 1.0).astype(jnp.bfloat16)[..., None],
        (B, S, 128))

    bspec = lambda shape: pl.BlockSpec((1,) + shape, lambda b, l: (b, 0, 0))
    espec = lambda shape: pl.BlockSpec(
        (1,) + shape, lambda b, l: (jnp.minimum(l, Le - 1), 0, 0))
    dspec = lambda shape: pl.BlockSpec(
        (1,) + shape, lambda b, l: (jnp.maximum(l - Le, 0), 0, 0))

    flops = (Le * B * (8 * S * C * C + 4 * S * S * C + 4 * S * C * F)
             + Ld * B * (12 * Nq * C * C + 4 * S * C * C + 4 * Nq * Nq * C
                         + 4 * Nq * S * C + 4 * Nq * C * F))
    trans = (Le * B * NH * (S * S + S)
             + Ld * B * NH * (Nq * Nq + Nq * S + 2 * Nq))
    wbytes = (Le * (2 * (4 * C * C + 2 * C * F) + 4 * (5 * C + F + 4 * C))
              + Ld * (2 * (8 * C * C + 2 * C * F) + 4 * (10 * C + F + 8 * C)))
    bytes_acc = 4 * B * (3 * S * C + Nq * C) + 4 * Ld * B * Nq * C + B * wbytes

    hid = pl.pallas_call(
        partial(_body, n_enc=Le),
        out_shape=jax.ShapeDtypeStruct((Ld, B, Nq, C), jnp.float32),
        grid=(B, L),
        in_specs=[
            bspec((S, C)),
            bspec((S, C)),
            bspec((S, 128)),
            pl.BlockSpec((Nq, C), lambda b, l: (0, 0)),
            espec((C, 3 * C)), espec((1, 3 * C)), espec((C, C)), espec((1, C)),
            espec((C, F)), espec((1, F)), espec((F, C)), espec((1, C)),
            espec((4, C)),
            dspec((C, 3 * C)), dspec((1, 3 * C)), dspec((C, C)), dspec((1, C)),
            dspec((C, 3 * C)), dspec((1, 3 * C)), dspec((C, C)), dspec((1, C)),
            dspec((C, F)), dspec((1, F)), dspec((F, C)), dspec((1, C)),
            dspec((6, C)),
            pl.BlockSpec((2, C), lambda b, l: (0, 0)),
        ],
        out_specs=pl.BlockSpec(
            (1, 1, Nq, C), lambda b, l: (jnp.maximum(l - Le, 0), b, 0, 0)),
        scratch_shapes=[
            pltpu.VMEM((S, C), jnp.float32),
            pltpu.VMEM((Nq, C), jnp.float32),
            pltpu.VMEM((S, C), jnp.float32),
            pltpu.VMEM((S, C), jnp.bfloat16),
            pltpu.VMEM((S, C), jnp.bfloat16),
        ],
        compiler_params=pltpu.CompilerParams(
            dimension_semantics=("arbitrary", "arbitrary")),
        cost_estimate=pl.CostEstimate(flops=flops, transcendentals=trans,
                                      bytes_accessed=bytes_acc),
    )(xt, post, valid, query_embed,
      e_wqkv, e_bqkv, e_wo, e_bo,
      e_ffn_w1, e_ffn_b1, e_ffn_w2, e_ffn_b2, e_ln,
      d_sa_wqkv, d_sa_bqkv, d_sa_wo, d_sa_bo,
      d_ca_wqkv, d_ca_bqkv, d_ca_wo, d_ca_bo,
      d_ffn_w1, d_ffn_b1, d_ffn_w2, d_ffn_b2, d_ln, dec_norm)

    hidden_state = jnp.transpose(hid, (0, 2, 1, 3))

    query_pos = jnp.broadcast_to(query_embed[None], (B, Nq, C))
    r = jnp.maximum(query_pos.reshape(B * Nq, C) @ ref_w1 + ref_b1, 0.0)
    r = r @ ref_w2 + ref_b2
    references = jax.nn.sigmoid(r).reshape(B, Nq, 2)

    return hidden_state, references
